# Initial kernel scaffold; baseline (speedup 1.0000x reference)
import jax
import jax.numpy as jnp
from jax.experimental import pallas as pl


def kernel(x, edge_index, edge_type, batch, W1, root1, b1, W2, root2, b2, Wc, bc):
    raise NotImplementedError("write your pallas kernel here")


# speedup vs baseline: 8.6636x; 8.6636x over previous
import functools

import jax
import jax.numpy as jnp
from jax import lax
from jax.experimental import pallas as pl
from jax.experimental.pallas import tpu as pltpu
from jax.experimental.pallas import tpu_sc as plsc

NC = 2
NS = 16
NW = NC * NS
L = 16

B = 128

NUM_GRAPHS = 64


def _ceil_to(a, m):
  return (a + m - 1) // m * m


def _sc_setup_body(n_nodes, n_rel, e_pad, nbins, dst_hbm, et_hbm, src_hbm,
                   w_hbm, gidx_hbm, cnt_sh, zbuf, ebuf0, ebuf1, ebuf2,
                   wbuf, gbuf, ones_v, recip_v):
  s_id = lax.axis_index("s")
  c_id = lax.axis_index("c")
  wid = s_id * NC + c_id

  bins_per_tile = nbins // NS
  def _z(i, _):
    zbuf[pl.ds(i * L, L)] = jnp.zeros((L,), jnp.float32)
    return 0
  lax.fori_loop(0, bins_per_tile // L, _z, 0)
  pltpu.sync_copy(zbuf, cnt_sh.at[pl.ds(s_id * bins_per_tile, bins_per_tile)])
  def _o(i, _):
    ones_v[pl.ds(i * L, L)] = jnp.ones((L,), jnp.float32)
    return 0
  lax.fori_loop(0, B // L, _o, 0)
  plsc.subcore_barrier()

  ept_cnt = e_pad // NS
  def _cnt(g, _):
    off = pl.multiple_of(s_id * ept_cnt + g * B, 8)
    pltpu.sync_copy(dst_hbm.at[pl.ds(off, B)], ebuf0)
    pltpu.sync_copy(et_hbm.at[pl.ds(off, B)], ebuf1)
    def _comp(i, _):
      dv = ebuf0[pl.ds(i * L, L)]
      ev = ebuf1[pl.ds(i * L, L)]
      ebuf2[pl.ds(i * L, L)] = dv * n_rel + ev
      return 0
    lax.fori_loop(0, B // L, _comp, 0)
    pltpu.sync_copy(ones_v, cnt_sh.at[ebuf2], add=True)
    return 0
  lax.fori_loop(0, ept_cnt // B, _cnt, 0)
  plsc.subcore_barrier()

  pltpu.sync_copy(cnt_sh.at[pl.ds(s_id * bins_per_tile, bins_per_tile)], zbuf)
  def _r(i, _):
    v = zbuf[pl.ds(i * L, L)]
    zbuf[pl.ds(i * L, L)] = 1.0 / jnp.maximum(v, 1.0)
    return 0
  lax.fori_loop(0, bins_per_tile // L, _r, 0)
  pltpu.sync_copy(zbuf, cnt_sh.at[pl.ds(s_id * bins_per_tile, bins_per_tile)])
  plsc.subcore_barrier()

  pltpu.sync_copy(cnt_sh, recip_v)
  ept = e_pad // NW
  def _w(g, _):
    off = pl.multiple_of(wid * ept + g * B, 8)
    pltpu.sync_copy(src_hbm.at[pl.ds(off, B)], ebuf0)
    pltpu.sync_copy(dst_hbm.at[pl.ds(off, B)], ebuf1)
    pltpu.sync_copy(et_hbm.at[pl.ds(off, B)], ebuf2)
    def _g(i, _):
      sv = ebuf0[pl.ds(i * L, L)]
      dv = ebuf1[pl.ds(i * L, L)]
      ev = ebuf2[pl.ds(i * L, L)]
      comp = dv * n_rel + ev
      wbuf[pl.ds(i * L, L)] = plsc.load_gather(recip_v, [comp])
      gbuf[pl.ds(i * L, L)] = ev * n_nodes + sv
      return 0
    lax.fori_loop(0, B // L, _g, 0)
    pltpu.sync_copy(wbuf, w_hbm.at[pl.ds(off, B)])
    pltpu.sync_copy(gbuf, gidx_hbm.at[pl.ds(off, B)])
    return 0
  lax.fori_loop(0, ept // B, _w, 0)


def _make_sc_setup(n_nodes, n_rel, e_pad, nbins):
  mesh = plsc.VectorSubcoreMesh(core_axis_name="c", subcore_axis_name="s")
  body = functools.partial(_sc_setup_body, n_nodes, n_rel, e_pad, nbins)
  return pl.kernel(
      body,
      out_type=(
          jax.ShapeDtypeStruct((e_pad,), jnp.float32),
          jax.ShapeDtypeStruct((e_pad,), jnp.int32),
      ),
      mesh=mesh,
      scratch_types=[
          pltpu.VMEM_SHARED((nbins,), jnp.float32),
          pltpu.VMEM((nbins // NS,), jnp.float32),
          pltpu.VMEM((B,), jnp.int32),
          pltpu.VMEM((B,), jnp.int32),
          pltpu.VMEM((B,), jnp.int32),
          pltpu.VMEM((B,), jnp.float32),
          pltpu.VMEM((B,), jnp.int32),
          pltpu.VMEM((B,), jnp.float32),
          pltpu.VMEM((nbins,), jnp.float32),
      ],
      compiler_params=pltpu.CompilerParams(needs_layout_passes=False),
      name="rgcn_sc_setup",
  )


def _sc_layer_body(n_acc, d, e_pad, xr_hbm, gidx_hbm, dst_hbm, w_hbm,
                   out_hbm, acc_sh, rows_v, gidx_v, dst_v, w_v, sem):
  s_id = lax.axis_index("s")
  c_id = lax.axis_index("c")
  wid = s_id * NC + c_id

  rows_per_tile = n_acc // NS
  def _z(i, _):
    for c8 in range(d // L):
      rows_v[i, pl.ds(c8 * L, L)] = jnp.zeros((L,), jnp.float32)
    return 0
  lax.fori_loop(0, B, _z, 0)
  for k in range(rows_per_tile // B):
    pltpu.sync_copy(rows_v, acc_sh.at[pl.ds(s_id * rows_per_tile + k * B, B)])
  plsc.subcore_barrier()

  ept = e_pad // NW
  def _edge(g, _):
    off = pl.multiple_of(wid * ept + g * B, 8)
    pltpu.sync_copy(gidx_hbm.at[pl.ds(off, B)], gidx_v)
    pltpu.sync_copy(dst_hbm.at[pl.ds(off, B)], dst_v)
    pltpu.sync_copy(w_hbm.at[pl.ds(off, B)], w_v)
    pltpu.async_copy(xr_hbm.at[gidx_v], rows_v, sem).wait()
    def _scale(j, _):
      wv = w_v[pl.ds(j * L, L)]
      for k in range(L):
        w = wv[k]
        i = j * L + k
        for c8 in range(d // L):
          rows_v[i, pl.ds(c8 * L, L)] = rows_v[i, pl.ds(c8 * L, L)] * w
      return 0
    lax.fori_loop(0, B // L, _scale, 0)
    pltpu.sync_copy(rows_v, acc_sh.at[dst_v], add=True)
    return 0
  lax.fori_loop(0, ept // B, _edge, 0)
  plsc.subcore_barrier()

  base = pl.multiple_of(c_id * n_acc + s_id * rows_per_tile, 8)
  pltpu.sync_copy(acc_sh.at[pl.ds(s_id * rows_per_tile, rows_per_tile)],
                  out_hbm.at[pl.ds(base, rows_per_tile)])


def _make_sc_layer(n_acc, d, e_pad):
  mesh = plsc.VectorSubcoreMesh(core_axis_name="c", subcore_axis_name="s")
  body = functools.partial(_sc_layer_body, n_acc, d, e_pad)
  return pl.kernel(
      body,
      out_type=jax.ShapeDtypeStruct((NC * n_acc, d), jnp.float32),
      mesh=mesh,
      scratch_types=[
          pltpu.VMEM_SHARED((n_acc, d), jnp.float32),
          pltpu.VMEM((B, d), jnp.float32),
          pltpu.VMEM((B,), jnp.int32),
          pltpu.VMEM((B,), jnp.int32),
          pltpu.VMEM((B,), jnp.float32),
          pltpu.SemaphoreType.DMA,
      ],
      name="rgcn_sc_layer",
  )


def _mm_body(x_ref, w_ref, o_ref):
  o_ref[0] = jnp.dot(x_ref[...], w_ref[0],
                     preferred_element_type=jnp.float32)


def _relation_matmul(x, w_stack, n_blk):
  n, d = x.shape
  r1 = w_stack.shape[0]
  grid = (r1, n // n_blk)
  return pl.pallas_call(
      _mm_body,
      grid=grid,
      in_specs=[
          pl.BlockSpec((n_blk, d), lambda r, i: (i, 0)),
          pl.BlockSpec((1, d, d), lambda r, i: (r, 0, 0)),
      ],
      out_specs=pl.BlockSpec((1, n_blk, d), lambda r, i: (r, i, 0)),
      out_shape=jax.ShapeDtypeStruct((r1, n, d), jnp.float32),
  )(x, w_stack)


def _combine_body(a0_ref, a1_ref, xr_ref, b_ref, o_ref):
  o_ref[...] = jnp.maximum(
      a0_ref[...] + a1_ref[...] + xr_ref[...] + b_ref[...], 0.0)


def _combine(acc0, acc1, xr_root, b, n_blk):
  n, d = acc0.shape
  grid = (n // n_blk,)
  return pl.pallas_call(
      _combine_body,
      grid=grid,
      in_specs=[
          pl.BlockSpec((n_blk, d), lambda i: (i, 0)),
          pl.BlockSpec((n_blk, d), lambda i: (i, 0)),
          pl.BlockSpec((n_blk, d), lambda i: (i, 0)),
          pl.BlockSpec((1, d), lambda i: (0, 0)),
      ],
      out_specs=pl.BlockSpec((n_blk, d), lambda i: (i, 0)),
      out_shape=jax.ShapeDtypeStruct((n, d), jnp.float32),
  )(acc0, acc1, xr_root, b.reshape(1, d))


def _pool_body(n_groups, h_ref, batch_ref, wc_ref, bc_ref, o_ref):
  npad = h_ref.shape[0]
  ids = lax.broadcasted_iota(jnp.int32, (n_groups, npad), 0)
  onehot = jnp.where(ids == batch_ref[...], 1.0, 0.0)
  sums = jnp.dot(onehot, h_ref[...], preferred_element_type=jnp.float32)
  cnt = jnp.sum(onehot, axis=1, keepdims=True)
  g = sums / jnp.maximum(cnt, 1.0)
  o_ref[...] = jnp.dot(g, wc_ref[...],
                       preferred_element_type=jnp.float32) + bc_ref[...]


def _pool_classify(h_pad, batch_pad, wc, bc, n_groups):
  npad, d = h_pad.shape
  c = wc.shape[1]
  return pl.pallas_call(
      functools.partial(_pool_body, n_groups),
      in_specs=[
          pl.BlockSpec((npad, d), lambda: (0, 0)),
          pl.BlockSpec((n_groups, npad), lambda: (0, 0)),
          pl.BlockSpec((d, c), lambda: (0, 0)),
          pl.BlockSpec((1, c), lambda: (0, 0)),
      ],
      out_specs=pl.BlockSpec((n_groups, c), lambda: (0, 0)),
      out_shape=jax.ShapeDtypeStruct((n_groups, c), jnp.float32),
  )(h_pad, jnp.broadcast_to(batch_pad[None, :], (n_groups, npad)), wc,
    bc.reshape(1, c))


def kernel(x, edge_index, edge_type, batch, W1, root1, b1, W2, root2, b2,
           Wc, bc):
  n, d = x.shape
  r = W1.shape[0]
  e = edge_index.shape[1]
  n_groups = NUM_GRAPHS

  src = edge_index[0].astype(jnp.int32)
  dst = edge_index[1].astype(jnp.int32)
  et = edge_type.astype(jnp.int32)
  batch32 = batch.astype(jnp.int32)

  e_pad = _ceil_to(e, NW * B)
  n_acc = _ceil_to(n + 1, NS * B)
  nbins = n_acc * r
  pad = e_pad - e

  src_p = jnp.concatenate([src, jnp.zeros((pad,), jnp.int32)])
  dst_p = jnp.concatenate([dst, jnp.full((pad,), n, jnp.int32)])
  et_p = jnp.concatenate([et, jnp.zeros((pad,), jnp.int32)])

  w_edge, gidx = _make_sc_setup(n, r, e_pad, nbins)(dst_p, et_p, src_p)

  sc_layer = _make_sc_layer(n_acc, d, e_pad)
  n_blk = 1000

  def layer(h, w_rel, root, b):
    w_stack = jnp.concatenate([w_rel, root[None]], axis=0)
    xr = _relation_matmul(h, w_stack, n_blk)
    xr_flat = xr[:r].reshape(r * n, d)
    acc = sc_layer(xr_flat, gidx, dst_p, w_edge)
    acc0 = acc[:n]
    acc1 = acc[n_acc:n_acc + n]
    return _combine(acc0, acc1, xr[r], b, n_blk)

  h = layer(x, W1, root1, b1)
  h = layer(h, W2, root2, b2)

  n_pad = _ceil_to(n, B)
  h_pad = jnp.pad(h, ((0, n_pad - n), (0, 0)))
  batch_pad = jnp.concatenate(
      [batch32, jnp.full((n_pad - n,), n_groups, jnp.int32)])
  return _pool_classify(h_pad, batch_pad, Wc, bc, n_groups)
